# baseline (speedup 1.0000x reference)
import functools

import jax
import jax.numpy as jnp
from jax import lax
from jax.experimental import pallas as pl
from jax.experimental.pallas import tpu as pltpu
from jax.experimental.pallas import tpu_sc as plsc

N_NODES = 10000
N_EDGES = 320000
D_FEAT = 128
D_EDGE = 16
D_HALF = D_FEAT // 2
D_ACC = D_HALF + 16

NC = 2
NS = 16
CHUNK = 80
NCHUNK = N_EDGES // (NS * CHUNK)
IBLK = 10
NBLK = NCHUNK // IBLK
N_PAD = 10240
ROWS_PER_TILE = N_PAD // NS


def _sc_segment_sums(xext, src3, dst3, ea5, z80, z16):
    mesh = plsc.VectorSubcoreMesh(core_axis_name="c", subcore_axis_name="s")

    @functools.partial(
        pl.kernel,
        out_type=[
            jax.ShapeDtypeStruct((NC, N_PAD, D_ACC), jnp.float32),
            jax.ShapeDtypeStruct((NC, N_PAD, D_EDGE), jnp.float32),
        ],
        mesh=mesh,
        compiler_params=pltpu.CompilerParams(use_tc_tiling_on_sc=False),
        scratch_types=[
            pltpu.VMEM((IBLK, CHUNK), jnp.int32),
            pltpu.VMEM((IBLK, CHUNK), jnp.int32),
            pltpu.VMEM((2, CHUNK, D_ACC), jnp.float32),
            pltpu.VMEM((2, CHUNK, D_ACC), jnp.float32),
            pltpu.VMEM((IBLK, CHUNK, D_EDGE), jnp.float32),
            pltpu.VMEM_SHARED((N_PAD, D_ACC), jnp.float32),
            pltpu.VMEM_SHARED((N_PAD, D_EDGE), jnp.float32),
            pltpu.SemaphoreType.DMA,
            pltpu.SemaphoreType.DMA,
            pltpu.SemaphoreType.DMA,
            pltpu.SemaphoreType.DMA,
        ],
    )
    def k(x_hbm, src_hbm, dst_hbm, ea_hbm, z80_hbm, z16_hbm,
          px_hbm, pe_hbm,
          src_v, dst_v, rows0, rows1, e_v, acc, acc_e, sem0, sem1,
          ssem0, ssem1):
        c = lax.axis_index("c")
        s = lax.axis_index("s")
        base = s * ROWS_PER_TILE
        pltpu.sync_copy(z80_hbm.at[pl.ds(base, ROWS_PER_TILE)],
                        acc.at[pl.ds(base, ROWS_PER_TILE)])
        pltpu.sync_copy(z16_hbm.at[pl.ds(base, ROWS_PER_TILE)],
                        acc_e.at[pl.ds(base, ROWS_PER_TILE)])
        plsc.subcore_barrier()

        bufs = (rows0, rows1)
        sems = (sem0, sem1)
        ssems = (ssem0, ssem1)
        NPAIR = IBLK // 2
        off16 = jnp.full((16,), N_NODES, jnp.int32)

        def fire_pair(p):
            for h in range(2):
                pltpu.async_copy(x_hbm.at[src_v.at[2 * p + h]],
                                 bufs[p % 2].at[h], sems[p % 2])

        def scatter_start(p, h):
            pltpu.async_copy(bufs[p % 2].at[h],
                             acc.at[dst_v.at[2 * p + h]],
                             ssems[p % 2], add=True)

        def scatter_wait(p, h):
            pltpu.make_async_copy(bufs[p % 2].at[h],
                                  acc.at[dst_v.at[2 * p + h]],
                                  ssems[p % 2]).wait()

        @pl.loop(0, NBLK)
        def _(ob):
            pltpu.sync_copy(src_hbm.at[s, ob], src_v)
            pltpu.sync_copy(dst_hbm.at[s, ob], dst_v)
            pltpu.sync_copy(ea_hbm.at[s, ob], e_v)

            @pl.when(c == 1)
            def _():
                @pl.loop(0, IBLK)
                def _(i):
                    for v in range(CHUNK // 16):
                        sl = pl.ds(v * 16, 16)
                        src_v[i, sl] = src_v[i, sl] + off16

            fire_pair(0)
            for p in range(NPAIR):
                buf, sem = bufs[p % 2], sems[p % 2]
                for h in range(2):
                    pltpu.make_async_copy(x_hbm.at[src_v.at[2 * p + h]],
                                          buf.at[h], sem).wait()
                if p + 1 < NPAIR:
                    if p >= 1:
                        for h in range(2):
                            scatter_wait(p - 1, h)
                    fire_pair(p + 1)
                for h in range(2):
                    jj = 2 * p + h
                    scatter_start(p, h)

                    @pl.when((jj % 2) == c)
                    def _():
                        pltpu.sync_copy(e_v.at[jj], acc_e.at[dst_v.at[jj]],
                                        add=True)
            for p in (NPAIR - 2, NPAIR - 1):
                for h in range(2):
                    scatter_wait(p, h)

        plsc.subcore_barrier()
        pltpu.sync_copy(acc.at[pl.ds(base, ROWS_PER_TILE)],
                        px_hbm.at[c, pl.ds(base, ROWS_PER_TILE)])
        pltpu.sync_copy(acc_e.at[pl.ds(base, ROWS_PER_TILE)],
                        pe_hbm.at[c, pl.ds(base, ROWS_PER_TILE)])

    return k(xext, src3, dst3, ea5, z80, z16)


def _tc_body(x_ref, px_ref, pe_ref, ws_ref, we_ref, wf_ref, b_ref, o_ref):
    deg = px_ref[0, :, D_HALF:D_HALF + 1]
    pe = pe_ref[0] + pe_ref[1]
    agg = (jnp.dot(px_ref[0, :, 0:D_HALF], ws_ref[0:D_HALF, :],
                   preferred_element_type=jnp.float32)
           + jnp.dot(px_ref[1, :, 0:D_HALF], ws_ref[D_HALF:D_FEAT, :],
                     preferred_element_type=jnp.float32)
           + jnp.dot(pe, we_ref[...], preferred_element_type=jnp.float32))
    agg = agg / jnp.maximum(deg, 1.0)
    h = jnp.dot(x_ref[...], wf_ref[...], preferred_element_type=jnp.float32)
    o_ref[...] = jnp.maximum(h + agg + b_ref[...], 0.0)


def kernel(x, edge_index, edge_attr, W_src, W_edge, W_self, b):
    src3 = edge_index[0].reshape(NS, NBLK, IBLK, CHUNK)
    dst3 = edge_index[1].reshape(NS, NBLK, IBLK, CHUNK)
    ea5 = edge_attr.reshape(NS, NBLK, IBLK, CHUNK, D_EDGE)
    xh = jnp.concatenate([x[:, :D_HALF], x[:, D_HALF:]], axis=0)
    xext = jnp.concatenate([xh, jnp.ones((NC * N_NODES, 16), jnp.float32)],
                           axis=1)
    z80 = jnp.zeros((N_PAD, D_ACC), jnp.float32)
    z16 = jnp.zeros((N_PAD, D_EDGE), jnp.float32)

    px, pe = _sc_segment_sums(xext, src3, dst3, ea5, z80, z16)

    R = 1000
    grid = (N_NODES // R,)
    out = pl.pallas_call(
        _tc_body,
        grid=grid,
        in_specs=[
            pl.BlockSpec((R, D_FEAT), lambda i: (i, 0)),
            pl.BlockSpec((NC, R, D_ACC), lambda i: (0, i, 0)),
            pl.BlockSpec((NC, R, D_EDGE), lambda i: (0, i, 0)),
            pl.BlockSpec((D_FEAT, D_FEAT), lambda i: (0, 0)),
            pl.BlockSpec((D_EDGE, D_FEAT), lambda i: (0, 0)),
            pl.BlockSpec((D_FEAT, D_FEAT), lambda i: (0, 0)),
            pl.BlockSpec((1, D_FEAT), lambda i: (0, 0)),
        ],
        out_specs=pl.BlockSpec((R, D_FEAT), lambda i: (i, 0)),
        out_shape=jax.ShapeDtypeStruct((N_NODES, D_FEAT), jnp.float32),
    )(x, px, pe, W_src, W_edge, W_self, b.reshape(1, D_FEAT))
    return out


# speedup vs baseline: 1.0086x; 1.0086x over previous
import functools

import jax
import jax.numpy as jnp
from jax import lax
from jax.experimental import pallas as pl
from jax.experimental.pallas import tpu as pltpu
from jax.experimental.pallas import tpu_sc as plsc

N_NODES = 10000
N_EDGES = 320000
D_FEAT = 128
D_EDGE = 16
D_HALF = D_FEAT // 2
D_ACC = D_HALF + 16

NC = 2
NS = 16
CHUNK = 80
NCHUNK = N_EDGES // (NS * CHUNK)
IBLK = 10
NBLK = NCHUNK // IBLK
N_PAD = 10240
ROWS_PER_TILE = N_PAD // NS


def _sc_segment_sums(xext, src3, dst3, ea, z80, z16):
    mesh = plsc.VectorSubcoreMesh(core_axis_name="c", subcore_axis_name="s")

    @functools.partial(
        pl.kernel,
        out_type=[
            jax.ShapeDtypeStruct((NC, N_PAD, D_ACC), jnp.float32),
            jax.ShapeDtypeStruct((NC, N_PAD, D_EDGE), jnp.float32),
        ],
        mesh=mesh,
        compiler_params=pltpu.CompilerParams(use_tc_tiling_on_sc=False),
        scratch_types=[
            pltpu.VMEM((IBLK, CHUNK), jnp.int32),
            pltpu.VMEM((IBLK, CHUNK), jnp.int32),
            pltpu.VMEM((2, CHUNK, D_ACC), jnp.float32),
            pltpu.VMEM((2, CHUNK, D_ACC), jnp.float32),
            pltpu.VMEM((IBLK * CHUNK, D_EDGE), jnp.float32),
            pltpu.VMEM_SHARED((N_PAD, D_ACC), jnp.float32),
            pltpu.VMEM_SHARED((N_PAD, D_EDGE), jnp.float32),
            pltpu.SemaphoreType.DMA,
            pltpu.SemaphoreType.DMA,
            pltpu.SemaphoreType.DMA,
            pltpu.SemaphoreType.DMA,
        ],
    )
    def k(x_hbm, src_hbm, dst_hbm, ea_hbm, z80_hbm, z16_hbm,
          px_hbm, pe_hbm,
          src_v, dst_v, rows0, rows1, e_v, acc, acc_e, sem0, sem1,
          ssem0, ssem1):
        c = lax.axis_index("c")
        s = lax.axis_index("s")
        base = s * ROWS_PER_TILE
        pltpu.sync_copy(z80_hbm.at[pl.ds(base, ROWS_PER_TILE)],
                        acc.at[pl.ds(base, ROWS_PER_TILE)])
        pltpu.sync_copy(z16_hbm.at[pl.ds(base, ROWS_PER_TILE)],
                        acc_e.at[pl.ds(base, ROWS_PER_TILE)])
        plsc.subcore_barrier()

        bufs = (rows0, rows1)
        sems = (sem0, sem1)
        ssems = (ssem0, ssem1)
        NPAIR = IBLK // 2
        off16 = jnp.full((16,), N_NODES, jnp.int32)

        def fire_pair(p):
            for h in range(2):
                pltpu.async_copy(x_hbm.at[src_v.at[2 * p + h]],
                                 bufs[p % 2].at[h], sems[p % 2])

        def scatter_start(p, h):
            pltpu.async_copy(bufs[p % 2].at[h],
                             acc.at[dst_v.at[2 * p + h]],
                             ssems[p % 2], add=True)

        def scatter_wait(p, h):
            pltpu.make_async_copy(bufs[p % 2].at[h],
                                  acc.at[dst_v.at[2 * p + h]],
                                  ssems[p % 2]).wait()

        @pl.loop(0, NBLK)
        def _(ob):
            pltpu.sync_copy(src_hbm.at[s, ob], src_v)
            pltpu.sync_copy(dst_hbm.at[s, ob], dst_v)
            pltpu.sync_copy(
                ea_hbm.at[pl.ds(s * (NCHUNK * CHUNK) + ob * (IBLK * CHUNK),
                                IBLK * CHUNK)], e_v)

            @pl.when(c == 1)
            def _():
                @pl.loop(0, IBLK)
                def _(i):
                    for v in range(CHUNK // 16):
                        sl = pl.ds(v * 16, 16)
                        src_v[i, sl] = src_v[i, sl] + off16

            fire_pair(0)
            for p in range(NPAIR):
                buf, sem = bufs[p % 2], sems[p % 2]
                for h in range(2):
                    pltpu.make_async_copy(x_hbm.at[src_v.at[2 * p + h]],
                                          buf.at[h], sem).wait()
                if p + 1 < NPAIR:
                    if p >= 1:
                        for h in range(2):
                            scatter_wait(p - 1, h)
                    fire_pair(p + 1)
                for h in range(2):
                    jj = 2 * p + h
                    scatter_start(p, h)

                    @pl.when((jj % 2) == c)
                    def _():
                        pltpu.sync_copy(e_v.at[pl.ds(jj * CHUNK, CHUNK)],
                                        acc_e.at[dst_v.at[jj]], add=True)
            for p in (NPAIR - 2, NPAIR - 1):
                for h in range(2):
                    scatter_wait(p, h)

        plsc.subcore_barrier()
        pltpu.sync_copy(acc.at[pl.ds(base, ROWS_PER_TILE)],
                        px_hbm.at[c, pl.ds(base, ROWS_PER_TILE)])
        pltpu.sync_copy(acc_e.at[pl.ds(base, ROWS_PER_TILE)],
                        pe_hbm.at[c, pl.ds(base, ROWS_PER_TILE)])

    return k(xext, src3, dst3, ea, z80, z16)


def _tc_body(x_ref, px_ref, pe_ref, ws_ref, we_ref, wf_ref, b_ref, o_ref):
    deg = px_ref[0, :, D_HALF:D_HALF + 1]
    pe = pe_ref[0] + pe_ref[1]
    agg = (jnp.dot(px_ref[0, :, 0:D_HALF], ws_ref[0:D_HALF, :],
                   preferred_element_type=jnp.float32)
           + jnp.dot(px_ref[1, :, 0:D_HALF], ws_ref[D_HALF:D_FEAT, :],
                     preferred_element_type=jnp.float32)
           + jnp.dot(pe, we_ref[...], preferred_element_type=jnp.float32))
    agg = agg / jnp.maximum(deg, 1.0)
    h = jnp.dot(x_ref[...], wf_ref[...], preferred_element_type=jnp.float32)
    o_ref[...] = jnp.maximum(h + agg + b_ref[...], 0.0)


def kernel(x, edge_index, edge_attr, W_src, W_edge, W_self, b):
    src3 = edge_index[0].reshape(NS, NBLK, IBLK, CHUNK)
    dst3 = edge_index[1].reshape(NS, NBLK, IBLK, CHUNK)
    xh = jnp.concatenate([x[:, :D_HALF], x[:, D_HALF:]], axis=0)
    xext = jnp.concatenate([xh, jnp.ones((NC * N_NODES, 16), jnp.float32)],
                           axis=1)
    z80 = jnp.zeros((N_PAD, D_ACC), jnp.float32)
    z16 = jnp.zeros((N_PAD, D_EDGE), jnp.float32)

    px, pe = _sc_segment_sums(xext, src3, dst3, edge_attr, z80, z16)

    R = 1000
    grid = (N_NODES // R,)
    out = pl.pallas_call(
        _tc_body,
        grid=grid,
        in_specs=[
            pl.BlockSpec((R, D_FEAT), lambda i: (i, 0)),
            pl.BlockSpec((NC, R, D_ACC), lambda i: (0, i, 0)),
            pl.BlockSpec((NC, R, D_EDGE), lambda i: (0, i, 0)),
            pl.BlockSpec((D_FEAT, D_FEAT), lambda i: (0, 0)),
            pl.BlockSpec((D_EDGE, D_FEAT), lambda i: (0, 0)),
            pl.BlockSpec((D_FEAT, D_FEAT), lambda i: (0, 0)),
            pl.BlockSpec((1, D_FEAT), lambda i: (0, 0)),
        ],
        out_specs=pl.BlockSpec((R, D_FEAT), lambda i: (i, 0)),
        out_shape=jax.ShapeDtypeStruct((N_NODES, D_FEAT), jnp.float32),
    )(x, px, pe, W_src, W_edge, W_self, b.reshape(1, D_FEAT))
    return out
